# baseline (speedup 1.0000x reference)
import jax
import jax.numpy as jnp
from jax import lax
from jax.experimental import pallas as pl
from jax.experimental.pallas import tpu as pltpu
from jax.experimental.pallas import tpu_sc as plsc

NUM_ITEMS = 1_000_000
ROWS = 16384
HIST = 200
N = ROWS * HIST

NC = 2
NS = 16
NW = NC * NS

RW = 32
W = RW * HIST
S_WIN = ROWS // NC // NS // RW
G_WIN = ROWS // NW // RW

INIT_BOUNCE = 7_808
INIT_WIN = 8
INIT_CHUNK = INIT_BOUNCE * INIT_WIN
INIT_TAIL = NUM_ITEMS - NS * INIT_CHUNK


def _hist_body(item_ref, counters_ref, ones_ref, zeros_ref,
               p0_ref, p1_ref, table,
               idx0, idx1, val_v, bnc0, bnc1,
               lsem0, lsem1, ssem0, ssem1, esem0, esem1, vsem):
    cid = lax.axis_index("c")
    sid = lax.axis_index("s")
    idxb, lsem = [idx0, idx1], [lsem0, lsem1]
    ssem, esem = [ssem0, ssem1], [esem0, esem1]
    bncb = [bnc0, bnc1]

    def fire_idx_rows(row0, b):
        for r in range(RW):
            pltpu.async_copy(item_ref.at[row0 + r],
                             idxb[b].at[pl.ds(r * HIST, HIST)], lsem[b])

    def wait_idx_rows(row0, b):
        for r in range(RW):
            pltpu.make_async_copy(item_ref.at[row0 + r],
                                  idxb[b].at[pl.ds(r * HIST, HIST)],
                                  lsem[b]).wait()

    def wait_sadd(b):
        pltpu.make_async_copy(val_v, table.at[idxb[b]], ssem[b]).wait()

    ibase = sid * INIT_CHUNK

    vdesc = pltpu.async_copy(ones_ref, val_v, vsem)

    @pl.when(cid == 0)
    def _init_from_counters():
        iloads = [None] * INIT_WIN
        istores = [None] * INIT_WIN
        iloads[0] = pltpu.async_copy(
            counters_ref.at[pl.ds(ibase, INIT_BOUNCE)], bnc0, lsem0)
        for j in range(INIT_WIN):
            iloads[j].wait()
            istores[j] = pltpu.async_copy(
                bncb[j % 2],
                table.at[pl.ds(ibase + j * INIT_BOUNCE, INIT_BOUNCE)],
                ssem[j % 2])
            if j >= 1:
                istores[j - 1].wait()
            if j + 1 < INIT_WIN:
                iloads[j + 1] = pltpu.async_copy(
                    counters_ref.at[pl.ds(ibase + (j + 1) * INIT_BOUNCE,
                                          INIT_BOUNCE)],
                    bncb[(j + 1) % 2], lsem[(j + 1) % 2])
        istores[INIT_WIN - 1].wait()

        @pl.when(sid == 0)
        def _tail():
            pltpu.sync_copy(counters_ref.at[pl.ds(NS * INIT_CHUNK,
                                                  INIT_TAIL)],
                            bnc0.at[pl.ds(0, INIT_TAIL)])
            pltpu.sync_copy(bnc0.at[pl.ds(0, INIT_TAIL)],
                            table.at[pl.ds(NS * INIT_CHUNK, INIT_TAIL)])

    @pl.when(cid == 1)
    def _init_zero():
        pltpu.sync_copy(zeros_ref, bnc0)
        for j in range(INIT_WIN):
            pltpu.sync_copy(
                bnc0, table.at[pl.ds(ibase + j * INIT_BOUNCE, INIT_BOUNCE)])

        @pl.when(sid == 0)
        def _tail():
            pltpu.sync_copy(bnc0.at[pl.ds(0, INIT_TAIL)],
                            table.at[pl.ds(NS * INIT_CHUNK, INIT_TAIL)])

    vdesc.wait()
    plsc.subcore_barrier()

    srow0 = (cid * NS + sid) * S_WIN * RW

    fire_idx_rows(srow0, 0)

    def sbody(k, carry):
        for b in range(2):
            w = 2 * k + b
            wait_idx_rows(srow0 + w * RW, b)
            pltpu.async_copy(val_v, table.at[idxb[b]], ssem[b], add=True)
            if b == 0:
                @pl.when(k > 0)
                def _():
                    wait_sadd(1)

                fire_idx_rows(srow0 + (w + 1) * RW, 1)
            else:
                wait_sadd(0)

                @pl.when(k < S_WIN // 2 - 1)
                def _():
                    fire_idx_rows(srow0 + (w + 1) * RW, 0)
        return carry

    lax.fori_loop(0, S_WIN // 2, sbody, 0)
    wait_sadd(1)

    plsc.subcore_barrier()

    def export(dst_ref):
        stores = [None] * INIT_WIN
        for j in range(INIT_WIN):
            b = j % 2
            if j >= 2:
                stores[j - 2].wait()
            pltpu.sync_copy(
                table.at[pl.ds(ibase + j * INIT_BOUNCE, INIT_BOUNCE)],
                bncb[b])
            stores[j] = pltpu.async_copy(
                bncb[b],
                dst_ref.at[pl.ds(ibase + j * INIT_BOUNCE, INIT_BOUNCE)],
                esem[b])
        stores[INIT_WIN - 2].wait()
        stores[INIT_WIN - 1].wait()

        @pl.when(sid == 0)
        def _tail():
            pltpu.sync_copy(table.at[pl.ds(NS * INIT_CHUNK, INIT_TAIL)],
                            bnc0.at[pl.ds(0, INIT_TAIL)])
            pltpu.async_copy(
                bnc0.at[pl.ds(0, INIT_TAIL)],
                dst_ref.at[pl.ds(NS * INIT_CHUNK, INIT_TAIL)],
                esem0).wait()

    @pl.when(cid == 0)
    def _export0():
        export(p0_ref)

    @pl.when(cid == 1)
    def _export1():
        export(p1_ref)


def _gather_body(item_ref, p0_ref, p1_ref, counters_ref, out_ref, table,
                 idx0, idx1, out0, out1, bnc0, bnc1, pb0, pb1, t16, c16,
                 lsem0, lsem1, ssem0, ssem1, osem0, osem1, gsem0, gsem1,
                 psem0, psem1):
    cid = lax.axis_index("c")
    sid = lax.axis_index("s")
    idxb, lsem = [idx0, idx1], [lsem0, lsem1]
    ssem = [ssem0, ssem1]
    outb, osem, gsem = [out0, out1], [osem0, osem1], [gsem0, gsem1]
    bncb, pbb, psem = [bnc0, bnc1], [pb0, pb1], [psem0, psem1]

    def fire_idx_rows(row0, b):
        for r in range(RW):
            pltpu.async_copy(item_ref.at[row0 + r],
                             idxb[b].at[pl.ds(r * HIST, HIST)], lsem[b])

    def wait_idx_rows(row0, b):
        for r in range(RW):
            pltpu.make_async_copy(item_ref.at[row0 + r],
                                  idxb[b].at[pl.ds(r * HIST, HIST)],
                                  lsem[b]).wait()

    def fire_out_rows(row0, b):
        for r in range(RW):
            pltpu.async_copy(outb[b].at[pl.ds(r * HIST, HIST)],
                             out_ref.at[row0 + r], osem[b])

    def wait_out_rows(row0, b):
        for r in range(RW):
            pltpu.make_async_copy(outb[b].at[pl.ds(r * HIST, HIST)],
                                  out_ref.at[row0 + r], osem[b]).wait()

    def merge_add(dst, src, nwords):
        def body(i, carry):
            for u in range(4):
                s = pl.ds(64 * i + 16 * u, 16)
                dst[s] = dst[s] + src[s]
            return carry
        lax.fori_loop(0, nwords // 64, body, 0)

    wid = sid * NC + cid
    grow0 = wid * G_WIN * RW
    fire_idx_rows(grow0, 0)
    fire_idx_rows(grow0 + RW, 1)

    islot = (sid + cid * (NS // 2)) % NS
    ibase = islot * INIT_CHUNK
    iloadsA = [None] * INIT_WIN
    iloadsB = [None] * INIT_WIN
    istores = [None] * INIT_WIN
    iloadsA[0] = pltpu.async_copy(
        p0_ref.at[pl.ds(ibase, INIT_BOUNCE)], bnc0, gsem0)
    iloadsB[0] = pltpu.async_copy(
        p1_ref.at[pl.ds(ibase, INIT_BOUNCE)], pb0, psem0)
    for j in range(INIT_WIN):
        b = j % 2
        iloadsA[j].wait()
        iloadsB[j].wait()
        if j >= 1:
            istores[j - 1].wait()
        if j + 1 < INIT_WIN:
            off = pl.ds(ibase + (j + 1) * INIT_BOUNCE, INIT_BOUNCE)
            iloadsA[j + 1] = pltpu.async_copy(
                p0_ref.at[off], bncb[1 - b], gsem[1 - b])
            iloadsB[j + 1] = pltpu.async_copy(
                p1_ref.at[off], pbb[1 - b], psem[1 - b])
        merge_add(bncb[b], pbb[b], INIT_BOUNCE)
        istores[j] = pltpu.async_copy(
            bncb[b],
            table.at[pl.ds(ibase + j * INIT_BOUNCE, INIT_BOUNCE)],
            ssem[b])
    istores[INIT_WIN - 1].wait()

    @pl.when(sid == 0)
    def _init_tail():
        pltpu.sync_copy(p0_ref.at[pl.ds(NS * INIT_CHUNK, INIT_TAIL)],
                        bnc0.at[pl.ds(0, INIT_TAIL)])
        pltpu.sync_copy(p1_ref.at[pl.ds(NS * INIT_CHUNK, INIT_TAIL)],
                        pb0.at[pl.ds(0, INIT_TAIL)])
        merge_add(bnc0, pb0, INIT_TAIL)
        pltpu.sync_copy(bnc0.at[pl.ds(0, INIT_TAIL)],
                        table.at[pl.ds(NS * INIT_CHUNK, INIT_TAIL)])

    plsc.subcore_barrier()

    @pl.when(sid == 0)
    def _fix_zero():
        pltpu.sync_copy(table.at[pl.ds(0, 16)], t16)
        pltpu.sync_copy(counters_ref.at[pl.ds(0, 16)], c16)
        lane = lax.iota(jnp.int32, 16)
        t16[...] = jnp.where(lane == 0, c16[...], t16[...])
        pltpu.sync_copy(t16, table.at[pl.ds(0, 16)])

    plsc.subcore_barrier()

    def gbody(k, carry):
        for b in range(2):
            w = 2 * k + b
            grow = grow0 + w * RW
            wait_idx_rows(grow, b)

            @pl.when(k > 0)
            def _():
                wait_out_rows(grow - 2 * RW, b)

            g = pltpu.async_copy(table.at[idxb[b]], outb[b], gsem[b])
            g.wait()
            fire_out_rows(grow, b)

            @pl.when(k < G_WIN // 2 - 1)
            def _():
                fire_idx_rows(grow + 2 * RW, b)
        return carry

    lax.fori_loop(0, G_WIN // 2, gbody, 0)
    wait_out_rows(grow0 + (G_WIN - 2) * RW, 0)
    wait_out_rows(grow0 + (G_WIN - 1) * RW, 1)


@jax.jit
def kernel(item, item_counters):
    mesh = plsc.VectorSubcoreMesh(core_axis_name="c", subcore_axis_name="s",
                                  num_cores=NC, num_subcores=NS)
    hist = pl.kernel(
        _hist_body,
        out_type=(jax.ShapeDtypeStruct((NUM_ITEMS,), jnp.float32),
                  jax.ShapeDtypeStruct((NUM_ITEMS,), jnp.float32)),
        mesh=mesh,
        scratch_types=[
            pltpu.VMEM_SHARED((NUM_ITEMS,), jnp.float32),
            pltpu.VMEM((W,), jnp.int32),
            pltpu.VMEM((W,), jnp.int32),
            pltpu.VMEM((W,), jnp.float32),
            pltpu.VMEM((INIT_BOUNCE,), jnp.float32),
            pltpu.VMEM((INIT_BOUNCE,), jnp.float32),
        ] + [pltpu.SemaphoreType.DMA] * 7,
        compiler_params=pltpu.CompilerParams(use_tc_tiling_on_sc=False),
    )
    gather = pl.kernel(
        _gather_body,
        out_type=jax.ShapeDtypeStruct((ROWS, HIST), jnp.float32),
        mesh=mesh,
        scratch_types=[
            pltpu.VMEM_SHARED((NUM_ITEMS,), jnp.float32),
            pltpu.VMEM((W,), jnp.int32),
            pltpu.VMEM((W,), jnp.int32),
            pltpu.VMEM((W,), jnp.float32),
            pltpu.VMEM((W,), jnp.float32),
            pltpu.VMEM((INIT_BOUNCE,), jnp.float32),
            pltpu.VMEM((INIT_BOUNCE,), jnp.float32),
            pltpu.VMEM((INIT_BOUNCE,), jnp.float32),
            pltpu.VMEM((INIT_BOUNCE,), jnp.float32),
            pltpu.VMEM((16,), jnp.float32),
            pltpu.VMEM((16,), jnp.float32),
        ] + [pltpu.SemaphoreType.DMA] * 10,
        compiler_params=pltpu.CompilerParams(use_tc_tiling_on_sc=False),
    )
    ones = jnp.ones((W,), jnp.float32)
    zeros = jnp.zeros((INIT_BOUNCE,), jnp.float32)
    p0, p1 = hist(item, item_counters, ones, zeros)
    return gather(item, p0, p1, item_counters)


# speedup vs baseline: 1.0119x; 1.0119x over previous
import jax
import jax.numpy as jnp
from jax import lax
from jax.experimental import pallas as pl
from jax.experimental.pallas import tpu as pltpu
from jax.experimental.pallas import tpu_sc as plsc

NUM_ITEMS = 1_000_000
ROWS = 16384
HIST = 200
N = ROWS * HIST

NC = 2
NS = 16
NW = NC * NS

RW = 32
W = RW * HIST
S_WIN = ROWS // NC // NS // RW
G_WIN = ROWS // NW // RW

INIT_BOUNCE = 7_808
INIT_WIN = 8
INIT_CHUNK = INIT_BOUNCE * INIT_WIN
INIT_TAIL = NUM_ITEMS - NS * INIT_CHUNK


def _hist_body(item_ref, counters_ref, ones_ref, zeros_ref,
               p0_ref, p1_ref, table,
               idx0, idx1, val_v, bnc0, bnc1,
               lsem0, lsem1, ssem0, ssem1, esem0, esem1, vsem):
    cid = lax.axis_index("c")
    sid = lax.axis_index("s")
    idxb, lsem = [idx0, idx1], [lsem0, lsem1]
    ssem, esem = [ssem0, ssem1], [esem0, esem1]
    bncb = [bnc0, bnc1]

    def fire_idx_rows(row0, b):
        for r in range(RW):
            pltpu.async_copy(item_ref.at[row0 + r],
                             idxb[b].at[pl.ds(r * HIST, HIST)], lsem[b])

    def wait_idx_rows(row0, b):
        for r in range(RW):
            pltpu.make_async_copy(item_ref.at[row0 + r],
                                  idxb[b].at[pl.ds(r * HIST, HIST)],
                                  lsem[b]).wait()

    def wait_sadd(b):
        pltpu.make_async_copy(val_v, table.at[idxb[b]], ssem[b]).wait()

    ibase = sid * INIT_CHUNK

    vdesc = pltpu.async_copy(ones_ref, val_v, vsem)

    @pl.when(cid == 0)
    def _init_from_counters():
        iloads = [None] * INIT_WIN
        istores = [None] * INIT_WIN
        iloads[0] = pltpu.async_copy(
            counters_ref.at[pl.ds(ibase, INIT_BOUNCE)], bnc0, lsem0)
        for j in range(INIT_WIN):
            iloads[j].wait()
            istores[j] = pltpu.async_copy(
                bncb[j % 2],
                table.at[pl.ds(ibase + j * INIT_BOUNCE, INIT_BOUNCE)],
                ssem[j % 2])
            if j >= 1:
                istores[j - 1].wait()
            if j + 1 < INIT_WIN:
                iloads[j + 1] = pltpu.async_copy(
                    counters_ref.at[pl.ds(ibase + (j + 1) * INIT_BOUNCE,
                                          INIT_BOUNCE)],
                    bncb[(j + 1) % 2], lsem[(j + 1) % 2])
        istores[INIT_WIN - 1].wait()

        @pl.when(sid == 0)
        def _tail():
            pltpu.sync_copy(counters_ref.at[pl.ds(NS * INIT_CHUNK,
                                                  INIT_TAIL)],
                            bnc0.at[pl.ds(0, INIT_TAIL)])
            pltpu.sync_copy(bnc0.at[pl.ds(0, INIT_TAIL)],
                            table.at[pl.ds(NS * INIT_CHUNK, INIT_TAIL)])

    @pl.when(cid == 1)
    def _init_zero():
        pltpu.sync_copy(zeros_ref, bnc0)
        for j in range(INIT_WIN):
            pltpu.sync_copy(
                bnc0, table.at[pl.ds(ibase + j * INIT_BOUNCE, INIT_BOUNCE)])

        @pl.when(sid == 0)
        def _tail():
            pltpu.sync_copy(bnc0.at[pl.ds(0, INIT_TAIL)],
                            table.at[pl.ds(NS * INIT_CHUNK, INIT_TAIL)])

    vdesc.wait()
    plsc.subcore_barrier()

    srow0 = (cid * NS + sid) * S_WIN * RW

    fire_idx_rows(srow0, 0)

    def sbody(k, carry):
        for b in range(2):
            w = 2 * k + b
            wait_idx_rows(srow0 + w * RW, b)
            pltpu.async_copy(val_v, table.at[idxb[b]], ssem[b], add=True)
            if b == 0:
                @pl.when(k > 0)
                def _():
                    wait_sadd(1)

                fire_idx_rows(srow0 + (w + 1) * RW, 1)
            else:
                wait_sadd(0)

                @pl.when(k < S_WIN // 2 - 1)
                def _():
                    fire_idx_rows(srow0 + (w + 1) * RW, 0)
        return carry

    lax.fori_loop(0, S_WIN // 2, sbody, 0)
    wait_sadd(1)

    plsc.subcore_barrier()

    def export(dst_ref):
        stores = [None] * INIT_WIN
        for j in range(INIT_WIN):
            b = j % 2
            if j >= 2:
                stores[j - 2].wait()
            pltpu.sync_copy(
                table.at[pl.ds(ibase + j * INIT_BOUNCE, INIT_BOUNCE)],
                bncb[b])
            stores[j] = pltpu.async_copy(
                bncb[b],
                dst_ref.at[pl.ds(ibase + j * INIT_BOUNCE, INIT_BOUNCE)],
                esem[b])
        stores[INIT_WIN - 2].wait()
        stores[INIT_WIN - 1].wait()

        @pl.when(sid == 0)
        def _tail():
            pltpu.sync_copy(table.at[pl.ds(NS * INIT_CHUNK, INIT_TAIL)],
                            bnc0.at[pl.ds(0, INIT_TAIL)])
            pltpu.async_copy(
                bnc0.at[pl.ds(0, INIT_TAIL)],
                dst_ref.at[pl.ds(NS * INIT_CHUNK, INIT_TAIL)],
                esem0).wait()

    @pl.when(cid == 0)
    def _export0():
        export(p0_ref)

    @pl.when(cid == 1)
    def _export1():
        export(p1_ref)


def _merge_body(p0_ref, p1_ref, o_ref):
    o_ref[...] = p0_ref[...] + p1_ref[...]


def _gather_body(item_ref, merged_ref, counters_ref, out_ref, table,
                 idx0, idx1, out0, out1, bnc0, bnc1, t16, c16,
                 lsem0, lsem1, ssem0, ssem1, osem0, osem1, gsem0, gsem1):
    cid = lax.axis_index("c")
    sid = lax.axis_index("s")
    idxb, lsem = [idx0, idx1], [lsem0, lsem1]
    ssem = [ssem0, ssem1]
    outb, osem, gsem = [out0, out1], [osem0, osem1], [gsem0, gsem1]
    bncb = [bnc0, bnc1]

    def fire_idx_rows(row0, b):
        for r in range(RW):
            pltpu.async_copy(item_ref.at[row0 + r],
                             idxb[b].at[pl.ds(r * HIST, HIST)], lsem[b])

    def wait_idx_rows(row0, b):
        for r in range(RW):
            pltpu.make_async_copy(item_ref.at[row0 + r],
                                  idxb[b].at[pl.ds(r * HIST, HIST)],
                                  lsem[b]).wait()

    def fire_out_rows(row0, b):
        for r in range(RW):
            pltpu.async_copy(outb[b].at[pl.ds(r * HIST, HIST)],
                             out_ref.at[row0 + r], osem[b])

    def wait_out_rows(row0, b):
        for r in range(RW):
            pltpu.make_async_copy(outb[b].at[pl.ds(r * HIST, HIST)],
                                  out_ref.at[row0 + r], osem[b]).wait()

    wid = sid * NC + cid
    grow0 = wid * G_WIN * RW
    fire_idx_rows(grow0, 0)
    fire_idx_rows(grow0 + RW, 1)

    islot = (sid + cid * (NS // 2)) % NS
    ibase = islot * INIT_CHUNK
    iloads = [None] * INIT_WIN
    istores = [None] * INIT_WIN
    iloads[0] = pltpu.async_copy(
        merged_ref.at[pl.ds(ibase, INIT_BOUNCE)], bnc0, gsem0)
    for j in range(INIT_WIN):
        iloads[j].wait()
        istores[j] = pltpu.async_copy(
            bncb[j % 2],
            table.at[pl.ds(ibase + j * INIT_BOUNCE, INIT_BOUNCE)],
            ssem[j % 2])
        if j >= 1:
            istores[j - 1].wait()
        if j + 1 < INIT_WIN:
            iloads[j + 1] = pltpu.async_copy(
                merged_ref.at[pl.ds(ibase + (j + 1) * INIT_BOUNCE,
                                    INIT_BOUNCE)],
                bncb[(j + 1) % 2], gsem[(j + 1) % 2])
    istores[INIT_WIN - 1].wait()

    @pl.when(sid == 0)
    def _init_tail():
        pltpu.sync_copy(merged_ref.at[pl.ds(NS * INIT_CHUNK, INIT_TAIL)],
                        bnc0.at[pl.ds(0, INIT_TAIL)])
        pltpu.sync_copy(bnc0.at[pl.ds(0, INIT_TAIL)],
                        table.at[pl.ds(NS * INIT_CHUNK, INIT_TAIL)])

    plsc.subcore_barrier()

    @pl.when(sid == 0)
    def _fix_zero():
        pltpu.sync_copy(table.at[pl.ds(0, 16)], t16)
        pltpu.sync_copy(counters_ref.at[pl.ds(0, 16)], c16)
        lane = lax.iota(jnp.int32, 16)
        t16[...] = jnp.where(lane == 0, c16[...], t16[...])
        pltpu.sync_copy(t16, table.at[pl.ds(0, 16)])

    plsc.subcore_barrier()

    def gbody(k, carry):
        for b in range(2):
            w = 2 * k + b
            grow = grow0 + w * RW
            wait_idx_rows(grow, b)

            @pl.when(k > 0)
            def _():
                wait_out_rows(grow - 2 * RW, b)

            g = pltpu.async_copy(table.at[idxb[b]], outb[b], gsem[b])
            g.wait()
            fire_out_rows(grow, b)

            @pl.when(k < G_WIN // 2 - 1)
            def _():
                fire_idx_rows(grow + 2 * RW, b)
        return carry

    lax.fori_loop(0, G_WIN // 2, gbody, 0)
    wait_out_rows(grow0 + (G_WIN - 2) * RW, 0)
    wait_out_rows(grow0 + (G_WIN - 1) * RW, 1)


@jax.jit
def kernel(item, item_counters):
    mesh = plsc.VectorSubcoreMesh(core_axis_name="c", subcore_axis_name="s",
                                  num_cores=NC, num_subcores=NS)
    hist = pl.kernel(
        _hist_body,
        out_type=(jax.ShapeDtypeStruct((NUM_ITEMS,), jnp.float32),
                  jax.ShapeDtypeStruct((NUM_ITEMS,), jnp.float32)),
        mesh=mesh,
        scratch_types=[
            pltpu.VMEM_SHARED((NUM_ITEMS,), jnp.float32),
            pltpu.VMEM((W,), jnp.int32),
            pltpu.VMEM((W,), jnp.int32),
            pltpu.VMEM((W,), jnp.float32),
            pltpu.VMEM((INIT_BOUNCE,), jnp.float32),
            pltpu.VMEM((INIT_BOUNCE,), jnp.float32),
        ] + [pltpu.SemaphoreType.DMA] * 7,
        compiler_params=pltpu.CompilerParams(use_tc_tiling_on_sc=False),
    )
    gather = pl.kernel(
        _gather_body,
        out_type=jax.ShapeDtypeStruct((ROWS, HIST), jnp.float32),
        mesh=mesh,
        scratch_types=[
            pltpu.VMEM_SHARED((NUM_ITEMS,), jnp.float32),
            pltpu.VMEM((W,), jnp.int32),
            pltpu.VMEM((W,), jnp.int32),
            pltpu.VMEM((W,), jnp.float32),
            pltpu.VMEM((W,), jnp.float32),
            pltpu.VMEM((INIT_BOUNCE,), jnp.float32),
            pltpu.VMEM((INIT_BOUNCE,), jnp.float32),
            pltpu.VMEM((16,), jnp.float32),
            pltpu.VMEM((16,), jnp.float32),
        ] + [pltpu.SemaphoreType.DMA] * 8,
        compiler_params=pltpu.CompilerParams(use_tc_tiling_on_sc=False),
    )
    ones = jnp.ones((W,), jnp.float32)
    zeros = jnp.zeros((INIT_BOUNCE,), jnp.float32)
    p0, p1 = hist(item, item_counters, ones, zeros)
    merged = pl.pallas_call(
        _merge_body,
        out_shape=jax.ShapeDtypeStruct((NUM_ITEMS,), jnp.float32),
    )(p0, p1)
    return gather(item, merged, item_counters)
